# Initial kernel scaffold; baseline (speedup 1.0000x reference)
import jax
import jax.numpy as jnp
from jax import lax
from jax.experimental import pallas as pl
from jax.experimental.pallas import tpu as pltpu
from jax.experimental.pallas import tpu_sc as plsc

_N, _E = 10000, 320000
_F, _H, _C = 128, 256, 121
_HALF = _H // 2
_NSUB = 16
_W = 80
_EPW_P = _E // _NSUB
_NW_P = _EPW_P // _W
_EPW_D = _E // (2 * _NSUB)
_NW_D = _EPW_D // _W
_RPS = _N // _NSUB
_BN_EPS = 1e-5

_BR = 1000
_GRID = _N // _BR

_sc_mesh = plsc.VectorSubcoreMesh(core_axis_name="c", subcore_axis_name="s")


def _deg_body(dst_hbm, z16_hbm, deg_hbm, ones_v, idx_v, acc):
    c = lax.axis_index("c")
    s = lax.axis_index("s")
    pltpu.sync_copy(z16_hbm.at[pl.ds(s * _RPS, _RPS)], acc.at[pl.ds(s * _RPS, _RPS)])
    one = jnp.ones((16,), jnp.float32)
    for r in range(_W):
        ones_v[r] = one
    plsc.subcore_barrier()

    def w_body(w, carry):
        base = c * (_E // 2) + s * _EPW_D + w * _W
        pltpu.sync_copy(dst_hbm.at[pl.ds(base, _W)], idx_v)
        pltpu.sync_copy(ones_v, acc.at[idx_v], add=True)
        return carry

    lax.fori_loop(0, _NW_D, w_body, 0)
    plsc.subcore_barrier()
    pltpu.sync_copy(acc.at[pl.ds(s * _RPS, _RPS)],
                    deg_hbm.at[c, pl.ds(s * _RPS, _RPS)])


def _sc_degree(dst, z16):
    return pl.kernel(
        _deg_body,
        out_type=jax.ShapeDtypeStruct((2, _N, 16), jnp.float32),
        mesh=_sc_mesh,
        scratch_types=[
            pltpu.VMEM((_W, 16), jnp.float32),
            pltpu.VMEM((_W,), jnp.int32),
            pltpu.VMEM_SHARED((_N, 16), jnp.float32),
        ],
    )(dst, z16)


def _prop_body(m0_hbm, m1_hbm, src_hbm, dst_hbm, zrow_hbm, s0_hbm, s1_hbm,
               idx_s, idx_d, gb, acc, sem):
    c = lax.axis_index("c")
    s = lax.axis_index("s")
    pltpu.sync_copy(zrow_hbm.at[pl.ds(s * _RPS, _RPS)],
                    acc.at[pl.ds(s * _RPS, _RPS)])
    plsc.subcore_barrier()

    def run(m_hbm, s_hbm):
        def w_body(w, carry):
            base = s * _EPW_P + w * _W
            pltpu.sync_copy(src_hbm.at[pl.ds(base, _W)], idx_s)
            pltpu.sync_copy(dst_hbm.at[pl.ds(base, _W)], idx_d)
            pltpu.async_copy(m_hbm.at[idx_s], gb, sem).wait()
            pltpu.sync_copy(gb, acc.at[idx_d], add=True)
            return carry

        lax.fori_loop(0, _NW_P, w_body, 0)
        plsc.subcore_barrier()
        pltpu.sync_copy(acc.at[pl.ds(s * _RPS, _RPS)],
                        s_hbm.at[pl.ds(s * _RPS, _RPS)])

    @pl.when(c == 0)
    def _():
        run(m0_hbm, s0_hbm)

    @pl.when(c == 1)
    def _():
        run(m1_hbm, s1_hbm)


def _sc_prop(m0, m1, src, dst, zrow):
    return pl.kernel(
        _prop_body,
        out_type=(jax.ShapeDtypeStruct((_N, _HALF), jnp.float32),
                  jax.ShapeDtypeStruct((_N, _HALF), jnp.float32)),
        mesh=_sc_mesh,
        scratch_types=[
            pltpu.VMEM((_W,), jnp.int32),
            pltpu.VMEM((_W,), jnp.int32),
            pltpu.VMEM((_W, _HALF), jnp.float32),
            pltpu.VMEM_SHARED((_N, _HALF), jnp.float32),
            pltpu.SemaphoreType.DMA,
        ],
    )(m0, m1, src, dst, zrow)


def _dinv_kernel(deg_ref, o_ref):
    d = deg_ref[...]
    deg = d[0] + d[1] + 1.0
    dinv = lax.rsqrt(deg)
    o_ref[...] = jnp.broadcast_to(dinv[:, :1], (_BR, _HALF))


def _tc_dinv(deg16):
    return pl.pallas_call(
        _dinv_kernel,
        grid=(_GRID,),
        in_specs=[pl.BlockSpec((2, _BR, 16), lambda i: (0, i, 0))],
        out_specs=pl.BlockSpec((_BR, _HALF), lambda i: (i, 0)),
        out_shape=jax.ShapeDtypeStruct((_N, _HALF), jnp.float32),
    )(deg16)


def _mm0_kernel(x_ref, w_ref, dv_ref, m0_ref, m1_ref):
    m = jnp.dot(x_ref[...], w_ref[...], preferred_element_type=jnp.float32)
    dv = dv_ref[...]
    m0_ref[...] = m[:, :_HALF] * dv
    m1_ref[...] = m[:, _HALF:] * dv


def _tc_mm0(x, w0, dinv):
    return pl.pallas_call(
        _mm0_kernel,
        grid=(_GRID,),
        in_specs=[
            pl.BlockSpec((_BR, _F), lambda i: (i, 0)),
            pl.BlockSpec((_F, _H), lambda i: (0, 0)),
            pl.BlockSpec((_BR, _HALF), lambda i: (i, 0)),
        ],
        out_specs=(pl.BlockSpec((_BR, _HALF), lambda i: (i, 0)),
                   pl.BlockSpec((_BR, _HALF), lambda i: (i, 0))),
        out_shape=(jax.ShapeDtypeStruct((_N, _HALF), jnp.float32),
                   jax.ShapeDtypeStruct((_N, _HALF), jnp.float32)),
    )(x, w0, dinv)


def _post_kernel(s0_ref, s1_ref, m0_ref, m1_ref, dv_ref, b_ref,
                 r_ref, st_ref, acc1, acc2):
    i = pl.program_id(0)
    dv = dv_ref[...]
    b = b_ref[...]
    p0 = (s0_ref[...] + m0_ref[...]) * dv + b[:, :_HALF]
    p1 = (s1_ref[...] + m1_ref[...]) * dv + b[:, _HALF:]
    r = jnp.concatenate([jnp.maximum(p0, 0.0), jnp.maximum(p1, 0.0)], axis=1)
    r_ref[...] = r

    @pl.when(i == 0)
    def _():
        acc1[...] = jnp.zeros_like(acc1)
        acc2[...] = jnp.zeros_like(acc2)

    acc1[...] += jnp.sum(r, axis=0, keepdims=True)
    acc2[...] += jnp.sum(r * r, axis=0, keepdims=True)

    @pl.when(i == _GRID - 1)
    def _():
        st_ref[...] = jnp.concatenate([acc1[...], acc2[...]], axis=0)


def _tc_post(s0, s1, m0, m1, dinv, b):
    return pl.pallas_call(
        _post_kernel,
        grid=(_GRID,),
        in_specs=[
            pl.BlockSpec((_BR, _HALF), lambda i: (i, 0)),
            pl.BlockSpec((_BR, _HALF), lambda i: (i, 0)),
            pl.BlockSpec((_BR, _HALF), lambda i: (i, 0)),
            pl.BlockSpec((_BR, _HALF), lambda i: (i, 0)),
            pl.BlockSpec((_BR, _HALF), lambda i: (i, 0)),
            pl.BlockSpec((1, _H), lambda i: (0, 0)),
        ],
        out_specs=(pl.BlockSpec((_BR, _H), lambda i: (i, 0)),
                   pl.BlockSpec((2, _H), lambda i: (0, 0))),
        out_shape=(jax.ShapeDtypeStruct((_N, _H), jnp.float32),
                   jax.ShapeDtypeStruct((2, _H), jnp.float32)),
        scratch_shapes=[pltpu.VMEM((1, _H), jnp.float32),
                        pltpu.VMEM((1, _H), jnp.float32)],
    )(s0, s1, m0, m1, dinv, b)


def _mm_kernel(r_ref, st_ref, w_ref, dv_ref, g_ref, bt_ref, m0_ref, m1_ref):
    st = st_ref[...]
    mu = st[0:1] / _N
    var = st[1:2] / _N - mu * mu
    a = g_ref[...] * lax.rsqrt(var + _BN_EPS)
    c = bt_ref[...] - mu * a
    rn = r_ref[...] * a + c
    m = jnp.dot(rn, w_ref[...], preferred_element_type=jnp.float32)
    dv = dv_ref[...]
    m0_ref[...] = m[:, :_HALF] * dv
    m1_ref[...] = m[:, _HALF:] * dv


def _tc_mm(r, st, w, dinv, g, bt):
    return pl.pallas_call(
        _mm_kernel,
        grid=(_GRID,),
        in_specs=[
            pl.BlockSpec((_BR, _H), lambda i: (i, 0)),
            pl.BlockSpec((2, _H), lambda i: (0, 0)),
            pl.BlockSpec((_H, _H), lambda i: (0, 0)),
            pl.BlockSpec((_BR, _HALF), lambda i: (i, 0)),
            pl.BlockSpec((1, _H), lambda i: (0, 0)),
            pl.BlockSpec((1, _H), lambda i: (0, 0)),
        ],
        out_specs=(pl.BlockSpec((_BR, _HALF), lambda i: (i, 0)),
                   pl.BlockSpec((_BR, _HALF), lambda i: (i, 0))),
        out_shape=(jax.ShapeDtypeStruct((_N, _HALF), jnp.float32),
                   jax.ShapeDtypeStruct((_N, _HALF), jnp.float32)),
    )(r, st, w, dinv, g, bt)


def _final_kernel(s0_ref, s1_ref, m0_ref, m1_ref, dv_ref, b_ref, o_ref):
    dv = dv_ref[...]
    b = b_ref[...]
    o0 = (s0_ref[...] + m0_ref[...]) * dv + b[:, :_HALF]
    o1 = (s1_ref[...] + m1_ref[...]) * dv + b[:, _HALF:]
    o_ref[...] = jnp.concatenate([o0, o1], axis=1)


def _tc_final(s0, s1, m0, m1, dinv, b):
    return pl.pallas_call(
        _final_kernel,
        grid=(_GRID,),
        in_specs=[
            pl.BlockSpec((_BR, _HALF), lambda i: (i, 0)),
            pl.BlockSpec((_BR, _HALF), lambda i: (i, 0)),
            pl.BlockSpec((_BR, _HALF), lambda i: (i, 0)),
            pl.BlockSpec((_BR, _HALF), lambda i: (i, 0)),
            pl.BlockSpec((_BR, _HALF), lambda i: (i, 0)),
            pl.BlockSpec((1, _H), lambda i: (0, 0)),
        ],
        out_specs=pl.BlockSpec((_BR, _H), lambda i: (i, 0)),
        out_shape=jax.ShapeDtypeStruct((_N, _H), jnp.float32),
    )(s0, s1, m0, m1, dinv, b)


def kernel(x, edge_index, W0, b0, g0, bt0, W1, b1, g1, bt1, W2, b2, g2, bt2,
           W3, b3, g3, bt3, Wh, bh):
    src = edge_index[0].astype(jnp.int32)
    dst = edge_index[1].astype(jnp.int32)
    z16 = jnp.zeros((_N, 16), jnp.float32)
    zrow = jnp.zeros((_N, _HALF), jnp.float32)

    deg16 = _sc_degree(dst, z16)
    dinv = _tc_dinv(deg16)

    m0, m1 = _tc_mm0(x, W0, dinv)
    layers = [(b0, g0, bt0, W1), (b1, g1, bt1, W2), (b2, g2, bt2, W3)]
    for b, g, bt, wnext in layers:
        s0, s1 = _sc_prop(m0, m1, src, dst, zrow)
        r, st = _tc_post(s0, s1, m0, m1, dinv, b.reshape(1, _H))
        m0, m1 = _tc_mm(r, st, wnext, dinv, g.reshape(1, _H), bt.reshape(1, _H))

    s0, s1 = _sc_prop(m0, m1, src, dst, zrow)
    r, st = _tc_post(s0, s1, m0, m1, dinv, b3.reshape(1, _H))
    wflat = jnp.transpose(Wh, (1, 0, 2)).reshape(_H, 2 * _C)
    wflat = jnp.pad(wflat, ((0, 0), (0, _H - 2 * _C)))
    m0, m1 = _tc_mm(r, st, wflat, dinv, g3.reshape(1, _H), bt3.reshape(1, _H))

    s0, s1 = _sc_prop(m0, m1, src, dst, zrow)
    bh_row = jnp.pad(bh.reshape(1, 2 * _C), ((0, 0), (0, _H - 2 * _C)))
    out = _tc_final(s0, s1, m0, m1, dinv, bh_row)

    return jnp.transpose(out[:, :2 * _C].reshape(_N, _C, 2), (1, 0, 2))


# speedup vs baseline: 6.6406x; 6.6406x over previous
import jax
import jax.numpy as jnp
from jax import lax
from jax.experimental import pallas as pl
from jax.experimental.pallas import tpu as pltpu
from jax.experimental.pallas import tpu_sc as plsc

_N, _E = 10000, 320000
_F, _H, _C = 128, 256, 121
_HALF = _H // 2
_NSUB = 16
_W = 80
_EPW_P = _E // _NSUB
_NW_P = _EPW_P // _W
_EPW_D = _E // (2 * _NSUB)
_NW_D = _EPW_D // _W
_RPS = 624
_TAIL_OFF = _RPS * _NSUB
_TAIL = _N - _TAIL_OFF
_BN_EPS = 1e-5

_BR = 1000
_GRID = _N // _BR

def _sc_mesh():
    return plsc.VectorSubcoreMesh(core_axis_name="c", subcore_axis_name="s",
                                  num_cores=2, num_subcores=_NSUB)


def _deg_body(dst_hbm, z16_hbm, deg_hbm, ones_v, idx_v, acc):
    c = lax.axis_index("c")
    s = lax.axis_index("s")
    pltpu.sync_copy(z16_hbm.at[pl.ds(s * _RPS, _RPS)], acc.at[pl.ds(s * _RPS, _RPS)])

    @pl.when(s == _NSUB - 1)
    def _():
        pltpu.sync_copy(z16_hbm.at[pl.ds(_TAIL_OFF, _TAIL)],
                        acc.at[pl.ds(_TAIL_OFF, _TAIL)])

    one = jnp.ones((16,), jnp.float32)
    for r in range(_W):
        ones_v[r] = one
    plsc.subcore_barrier()

    def w_body(w, carry):
        base = c * (_E // 2) + s * _EPW_D + w * _W
        pltpu.sync_copy(dst_hbm.at[pl.ds(base, _W)], idx_v)
        pltpu.sync_copy(ones_v, acc.at[idx_v], add=True)
        return carry

    lax.fori_loop(0, _NW_D, w_body, 0)
    plsc.subcore_barrier()
    pltpu.sync_copy(acc.at[pl.ds(s * _RPS, _RPS)],
                    deg_hbm.at[c, pl.ds(s * _RPS, _RPS)])

    @pl.when(s == _NSUB - 1)
    def _():
        pltpu.sync_copy(acc.at[pl.ds(_TAIL_OFF, _TAIL)],
                        deg_hbm.at[c, pl.ds(_TAIL_OFF, _TAIL)])


def _sc_degree(dst, z16):
    return pl.kernel(
        _deg_body,
        out_type=jax.ShapeDtypeStruct((2, _N, 16), jnp.float32),
        mesh=_sc_mesh(),
        scratch_types=[
            pltpu.VMEM((_W, 16), jnp.float32),
            pltpu.VMEM((_W,), jnp.int32),
            pltpu.VMEM_SHARED((_N, 16), jnp.float32),
        ],
    )(dst, z16)


def _prop_body(m0_hbm, m1_hbm, src_hbm, dst_hbm, zrow_hbm, s0_hbm, s1_hbm,
               idx_s, idx_d, gb, acc, sem):
    c = lax.axis_index("c")
    s = lax.axis_index("s")
    pltpu.sync_copy(zrow_hbm.at[pl.ds(s * _RPS, _RPS)],
                    acc.at[pl.ds(s * _RPS, _RPS)])

    @pl.when(s == _NSUB - 1)
    def _():
        pltpu.sync_copy(zrow_hbm.at[pl.ds(_TAIL_OFF, _TAIL)],
                        acc.at[pl.ds(_TAIL_OFF, _TAIL)])

    plsc.subcore_barrier()

    def run(m_hbm, s_hbm):
        def w_body(w, carry):
            base = s * _EPW_P + w * _W
            pltpu.sync_copy(src_hbm.at[pl.ds(base, _W)], idx_s)
            pltpu.sync_copy(dst_hbm.at[pl.ds(base, _W)], idx_d)
            pltpu.async_copy(m_hbm.at[idx_s], gb, sem).wait()
            pltpu.sync_copy(gb, acc.at[idx_d], add=True)
            return carry

        lax.fori_loop(0, _NW_P, w_body, 0)
        plsc.subcore_barrier()
        pltpu.sync_copy(acc.at[pl.ds(s * _RPS, _RPS)],
                        s_hbm.at[pl.ds(s * _RPS, _RPS)])

        @pl.when(s == _NSUB - 1)
        def _():
            pltpu.sync_copy(acc.at[pl.ds(_TAIL_OFF, _TAIL)],
                            s_hbm.at[pl.ds(_TAIL_OFF, _TAIL)])

    @pl.when(c == 0)
    def _():
        run(m0_hbm, s0_hbm)

    @pl.when(c == 1)
    def _():
        run(m1_hbm, s1_hbm)


def _sc_prop(m0, m1, src, dst, zrow):
    return pl.kernel(
        _prop_body,
        out_type=(jax.ShapeDtypeStruct((_N, _HALF), jnp.float32),
                  jax.ShapeDtypeStruct((_N, _HALF), jnp.float32)),
        mesh=_sc_mesh(),
        scratch_types=[
            pltpu.VMEM((_W,), jnp.int32),
            pltpu.VMEM((_W,), jnp.int32),
            pltpu.VMEM((_W, _HALF), jnp.float32),
            pltpu.VMEM_SHARED((_N, _HALF), jnp.float32),
            pltpu.SemaphoreType.DMA,
        ],
    )(m0, m1, src, dst, zrow)


def _dinv_kernel(deg_ref, o_ref):
    d = deg_ref[...]
    deg = d[0] + d[1] + 1.0
    dinv = lax.rsqrt(deg)
    o_ref[...] = jnp.broadcast_to(dinv[:, :1], (_BR, _HALF))


def _tc_dinv(deg16):
    return pl.pallas_call(
        _dinv_kernel,
        grid=(_GRID,),
        in_specs=[pl.BlockSpec((2, _BR, 16), lambda i: (0, i, 0))],
        out_specs=pl.BlockSpec((_BR, _HALF), lambda i: (i, 0)),
        out_shape=jax.ShapeDtypeStruct((_N, _HALF), jnp.float32),
    )(deg16)


def _mm0_kernel(x_ref, w_ref, dv_ref, m0_ref, m1_ref):
    m = jnp.dot(x_ref[...], w_ref[...], preferred_element_type=jnp.float32)
    dv = dv_ref[...]
    m0_ref[...] = m[:, :_HALF] * dv
    m1_ref[...] = m[:, _HALF:] * dv


def _tc_mm0(x, w0, dinv):
    return pl.pallas_call(
        _mm0_kernel,
        grid=(_GRID,),
        in_specs=[
            pl.BlockSpec((_BR, _F), lambda i: (i, 0)),
            pl.BlockSpec((_F, _H), lambda i: (0, 0)),
            pl.BlockSpec((_BR, _HALF), lambda i: (i, 0)),
        ],
        out_specs=(pl.BlockSpec((_BR, _HALF), lambda i: (i, 0)),
                   pl.BlockSpec((_BR, _HALF), lambda i: (i, 0))),
        out_shape=(jax.ShapeDtypeStruct((_N, _HALF), jnp.float32),
                   jax.ShapeDtypeStruct((_N, _HALF), jnp.float32)),
    )(x, w0, dinv)


def _post_kernel(s0_ref, s1_ref, m0_ref, m1_ref, dv_ref, b_ref,
                 r_ref, st_ref, acc1, acc2):
    i = pl.program_id(0)
    dv = dv_ref[...]
    b = b_ref[...]
    p0 = (s0_ref[...] + m0_ref[...]) * dv + b[:, :_HALF]
    p1 = (s1_ref[...] + m1_ref[...]) * dv + b[:, _HALF:]
    r = jnp.concatenate([jnp.maximum(p0, 0.0), jnp.maximum(p1, 0.0)], axis=1)
    r_ref[...] = r

    @pl.when(i == 0)
    def _():
        acc1[...] = jnp.zeros_like(acc1)
        acc2[...] = jnp.zeros_like(acc2)

    acc1[...] += jnp.sum(r, axis=0, keepdims=True)
    acc2[...] += jnp.sum(r * r, axis=0, keepdims=True)

    @pl.when(i == _GRID - 1)
    def _():
        st_ref[...] = jnp.concatenate([acc1[...], acc2[...]], axis=0)


def _tc_post(s0, s1, m0, m1, dinv, b):
    return pl.pallas_call(
        _post_kernel,
        grid=(_GRID,),
        in_specs=[
            pl.BlockSpec((_BR, _HALF), lambda i: (i, 0)),
            pl.BlockSpec((_BR, _HALF), lambda i: (i, 0)),
            pl.BlockSpec((_BR, _HALF), lambda i: (i, 0)),
            pl.BlockSpec((_BR, _HALF), lambda i: (i, 0)),
            pl.BlockSpec((_BR, _HALF), lambda i: (i, 0)),
            pl.BlockSpec((1, _H), lambda i: (0, 0)),
        ],
        out_specs=(pl.BlockSpec((_BR, _H), lambda i: (i, 0)),
                   pl.BlockSpec((2, _H), lambda i: (0, 0))),
        out_shape=(jax.ShapeDtypeStruct((_N, _H), jnp.float32),
                   jax.ShapeDtypeStruct((2, _H), jnp.float32)),
        scratch_shapes=[pltpu.VMEM((1, _H), jnp.float32),
                        pltpu.VMEM((1, _H), jnp.float32)],
    )(s0, s1, m0, m1, dinv, b)


def _mm_kernel(r_ref, st_ref, w_ref, dv_ref, g_ref, bt_ref, m0_ref, m1_ref):
    st = st_ref[...]
    mu = st[0:1] / _N
    var = st[1:2] / _N - mu * mu
    a = g_ref[...] * lax.rsqrt(var + _BN_EPS)
    c = bt_ref[...] - mu * a
    rn = r_ref[...] * a + c
    m = jnp.dot(rn, w_ref[...], preferred_element_type=jnp.float32)
    dv = dv_ref[...]
    m0_ref[...] = m[:, :_HALF] * dv
    m1_ref[...] = m[:, _HALF:] * dv


def _tc_mm(r, st, w, dinv, g, bt):
    return pl.pallas_call(
        _mm_kernel,
        grid=(_GRID,),
        in_specs=[
            pl.BlockSpec((_BR, _H), lambda i: (i, 0)),
            pl.BlockSpec((2, _H), lambda i: (0, 0)),
            pl.BlockSpec((_H, _H), lambda i: (0, 0)),
            pl.BlockSpec((_BR, _HALF), lambda i: (i, 0)),
            pl.BlockSpec((1, _H), lambda i: (0, 0)),
            pl.BlockSpec((1, _H), lambda i: (0, 0)),
        ],
        out_specs=(pl.BlockSpec((_BR, _HALF), lambda i: (i, 0)),
                   pl.BlockSpec((_BR, _HALF), lambda i: (i, 0))),
        out_shape=(jax.ShapeDtypeStruct((_N, _HALF), jnp.float32),
                   jax.ShapeDtypeStruct((_N, _HALF), jnp.float32)),
    )(r, st, w, dinv, g, bt)


def _final_kernel(s0_ref, s1_ref, m0_ref, m1_ref, dv_ref, b_ref, o_ref):
    dv = dv_ref[...]
    b = b_ref[...]
    o0 = (s0_ref[...] + m0_ref[...]) * dv + b[:, :_HALF]
    o1 = (s1_ref[...] + m1_ref[...]) * dv + b[:, _HALF:]
    o_ref[...] = jnp.concatenate([o0, o1], axis=1)


def _tc_final(s0, s1, m0, m1, dinv, b):
    return pl.pallas_call(
        _final_kernel,
        grid=(_GRID,),
        in_specs=[
            pl.BlockSpec((_BR, _HALF), lambda i: (i, 0)),
            pl.BlockSpec((_BR, _HALF), lambda i: (i, 0)),
            pl.BlockSpec((_BR, _HALF), lambda i: (i, 0)),
            pl.BlockSpec((_BR, _HALF), lambda i: (i, 0)),
            pl.BlockSpec((_BR, _HALF), lambda i: (i, 0)),
            pl.BlockSpec((1, _H), lambda i: (0, 0)),
        ],
        out_specs=pl.BlockSpec((_BR, _H), lambda i: (i, 0)),
        out_shape=jax.ShapeDtypeStruct((_N, _H), jnp.float32),
    )(s0, s1, m0, m1, dinv, b)


def kernel(x, edge_index, W0, b0, g0, bt0, W1, b1, g1, bt1, W2, b2, g2, bt2,
           W3, b3, g3, bt3, Wh, bh):
    src = edge_index[0].astype(jnp.int32)
    dst = edge_index[1].astype(jnp.int32)
    z16 = jnp.zeros((_N, 16), jnp.float32)
    zrow = jnp.zeros((_N, _HALF), jnp.float32)

    deg16 = _sc_degree(dst, z16)
    dinv = _tc_dinv(deg16)

    m0, m1 = _tc_mm0(x, W0, dinv)
    layers = [(b0, g0, bt0, W1), (b1, g1, bt1, W2), (b2, g2, bt2, W3)]
    for b, g, bt, wnext in layers:
        s0, s1 = _sc_prop(m0, m1, src, dst, zrow)
        r, st = _tc_post(s0, s1, m0, m1, dinv, b.reshape(1, _H))
        m0, m1 = _tc_mm(r, st, wnext, dinv, g.reshape(1, _H), bt.reshape(1, _H))

    s0, s1 = _sc_prop(m0, m1, src, dst, zrow)
    r, st = _tc_post(s0, s1, m0, m1, dinv, b3.reshape(1, _H))
    wflat = jnp.transpose(Wh, (1, 0, 2)).reshape(_H, 2 * _C)
    wflat = jnp.pad(wflat, ((0, 0), (0, _H - 2 * _C)))
    m0, m1 = _tc_mm(r, st, wflat, dinv, g3.reshape(1, _H), bt3.reshape(1, _H))

    s0, s1 = _sc_prop(m0, m1, src, dst, zrow)
    bh_row = jnp.pad(bh.reshape(1, 2 * _C), ((0, 0), (0, _H - 2 * _C)))
    out = _tc_final(s0, s1, m0, m1, dinv, bh_row)

    return jnp.transpose(out[:, :2 * _C].reshape(_N, _C, 2), (1, 0, 2))


# speedup vs baseline: 17.4245x; 2.6239x over previous
import jax
import jax.numpy as jnp
from jax import lax
from jax.experimental import pallas as pl
from jax.experimental.pallas import tpu as pltpu
from jax.experimental.pallas import tpu_sc as plsc

_N, _E = 10000, 320000
_F, _H, _C = 128, 256, 121
_HALF = _H // 2
_NSUB = 16
_W = 125
_EPW_P = _E // _NSUB
_NW_P = _EPW_P // _W
_RCH = 16
_NCH = _NW_P // _RCH
_WD = 80
_EPW_D = _E // (2 * _NSUB)
_NW_D = _EPW_D // _WD
_RPS = 624
_TAIL_OFF = _RPS * _NSUB
_TAIL = _N - _TAIL_OFF
_BN_EPS = 1e-5

_BR = 1000
_GRID = _N // _BR

def _sc_mesh():
    return plsc.VectorSubcoreMesh(core_axis_name="c", subcore_axis_name="s",
                                  num_cores=2, num_subcores=_NSUB)


def _deg_body(dst_hbm, z16_hbm, deg_hbm, ones_v, idx_v, acc):
    c = lax.axis_index("c")
    s = lax.axis_index("s")
    pltpu.sync_copy(z16_hbm.at[pl.ds(s * _RPS, _RPS)], acc.at[pl.ds(s * _RPS, _RPS)])

    @pl.when(s == _NSUB - 1)
    def _():
        pltpu.sync_copy(z16_hbm.at[pl.ds(_TAIL_OFF, _TAIL)],
                        acc.at[pl.ds(_TAIL_OFF, _TAIL)])

    one = jnp.ones((16,), jnp.float32)
    for r in range(_WD):
        ones_v[r] = one
    pltpu.sync_copy(dst_hbm.at[c, s], idx_v)
    plsc.subcore_barrier()

    def w_body(w, carry):
        pltpu.sync_copy(ones_v, acc.at[idx_v.at[w]], add=True)
        return carry

    lax.fori_loop(0, _NW_D, w_body, 0)
    plsc.subcore_barrier()
    pltpu.sync_copy(acc.at[pl.ds(s * _RPS, _RPS)],
                    deg_hbm.at[c, pl.ds(s * _RPS, _RPS)])

    @pl.when(s == _NSUB - 1)
    def _():
        pltpu.sync_copy(acc.at[pl.ds(_TAIL_OFF, _TAIL)],
                        deg_hbm.at[c, pl.ds(_TAIL_OFF, _TAIL)])


def _sc_degree(dst4, z16):
    return pl.kernel(
        _deg_body,
        out_type=jax.ShapeDtypeStruct((2, _N, 16), jnp.float32),
        mesh=_sc_mesh(),
        scratch_types=[
            pltpu.VMEM((_WD, 16), jnp.float32),
            pltpu.VMEM((_NW_D, _WD), jnp.int32),
            pltpu.VMEM_SHARED((_N, 16), jnp.float32),
        ],
    )(dst4, z16)


def _prop_body(m0_hbm, m1_hbm, src4_hbm, dst4_hbm, zrow_hbm, s0_hbm, s1_hbm,
               is0, is1, id0, id1, gb0, gb1, acc,
               semg0, semg1, sems0, sems1, semi0, semi1):
    c = lax.axis_index("c")
    s = lax.axis_index("s")
    pltpu.sync_copy(zrow_hbm.at[pl.ds(s * _RPS, _RPS)],
                    acc.at[pl.ds(s * _RPS, _RPS)])

    @pl.when(s == _NSUB - 1)
    def _():
        pltpu.sync_copy(zrow_hbm.at[pl.ds(_TAIL_OFF, _TAIL)],
                        acc.at[pl.ds(_TAIL_OFF, _TAIL)])

    pltpu.async_copy(src4_hbm.at[s, 0], is0, semi0)
    pltpu.async_copy(dst4_hbm.at[s, 0], id0, semi0)
    plsc.subcore_barrier()

    islot = ((is0, id0, semi0), (is1, id1, semi1))
    gslot = ((gb0, semg0, sems0), (gb1, semg1, sems1))

    def run(m_hbm, s_hbm):
        pltpu.make_async_copy(src4_hbm.at[s, 0], is0, semi0).wait()
        pltpu.make_async_copy(dst4_hbm.at[s, 0], id0, semi0).wait()
        pltpu.async_copy(m_hbm.at[is0.at[0]], gb0, semg0)

        def outer(q, carry):
            for cs in range(2):
                k = q * 2 + cs
                isrc, idst, semi = islot[cs]
                oisrc, oidst, osemi = islot[1 - cs]

                def win(j, cc):
                    for b in range(2):
                        r = j * 2 + b
                        w = k * _RCH + r
                        gb, semg, sems = gslot[b]
                        ogb, osemg, osems = gslot[1 - b]

                        @pl.when(w >= 1)
                        def _():
                            pltpu.make_async_copy(
                                ogb, acc.at[idst.at[0]], osems).wait()

                        if b == 0:
                            @pl.when(jnp.logical_and(j == 0, k + 1 < _NCH))
                            def _():
                                pltpu.async_copy(src4_hbm.at[s, k + 1],
                                                 oisrc, osemi)
                                pltpu.async_copy(dst4_hbm.at[s, k + 1],
                                                 oidst, osemi)

                        @pl.when(r < _RCH - 1)
                        def _():
                            pltpu.async_copy(m_hbm.at[isrc.at[r + 1]],
                                             ogb, osemg)

                        @pl.when(jnp.logical_and(r == _RCH - 1,
                                                 w + 1 < _NW_P))
                        def _():
                            pltpu.make_async_copy(
                                src4_hbm.at[s, k], oisrc, osemi).wait()
                            pltpu.make_async_copy(
                                dst4_hbm.at[s, k], oidst, osemi).wait()
                            pltpu.async_copy(m_hbm.at[oisrc.at[0]],
                                             ogb, osemg)

                        pltpu.make_async_copy(m_hbm.at[isrc.at[r]],
                                              gb, semg).wait()
                        pltpu.async_copy(gb, acc.at[idst.at[r]],
                                         sems, add=True)
                    return cc

                lax.fori_loop(0, _RCH // 2, win, 0)
            return carry

        lax.fori_loop(0, _NCH // 2, outer, 0)
        pltpu.make_async_copy(gb1, acc.at[id0.at[0]], sems1).wait()
        plsc.subcore_barrier()
        pltpu.sync_copy(acc.at[pl.ds(s * _RPS, _RPS)],
                        s_hbm.at[pl.ds(s * _RPS, _RPS)])

        @pl.when(s == _NSUB - 1)
        def _():
            pltpu.sync_copy(acc.at[pl.ds(_TAIL_OFF, _TAIL)],
                            s_hbm.at[pl.ds(_TAIL_OFF, _TAIL)])

    @pl.when(c == 0)
    def _():
        run(m0_hbm, s0_hbm)

    @pl.when(c == 1)
    def _():
        run(m1_hbm, s1_hbm)


def _sc_prop(m0, m1, src4, dst4, zrow):
    return pl.kernel(
        _prop_body,
        out_type=(jax.ShapeDtypeStruct((_N, _HALF), jnp.float32),
                  jax.ShapeDtypeStruct((_N, _HALF), jnp.float32)),
        mesh=_sc_mesh(),
        scratch_types=[
            pltpu.VMEM((_RCH, _W), jnp.int32),
            pltpu.VMEM((_RCH, _W), jnp.int32),
            pltpu.VMEM((_RCH, _W), jnp.int32),
            pltpu.VMEM((_RCH, _W), jnp.int32),
            pltpu.VMEM((_W, _HALF), jnp.float32),
            pltpu.VMEM((_W, _HALF), jnp.float32),
            pltpu.VMEM_SHARED((_N, _HALF), jnp.float32),
            pltpu.SemaphoreType.DMA,
            pltpu.SemaphoreType.DMA,
            pltpu.SemaphoreType.DMA,
            pltpu.SemaphoreType.DMA,
            pltpu.SemaphoreType.DMA,
            pltpu.SemaphoreType.DMA,
        ],
    )(m0, m1, src4, dst4, zrow)


def _dinv_kernel(deg_ref, o_ref):
    d = deg_ref[...]
    deg = d[0] + d[1] + 1.0
    dinv = lax.rsqrt(deg)
    o_ref[...] = jnp.broadcast_to(dinv[:, :1], (_BR, _HALF))


def _tc_dinv(deg16):
    return pl.pallas_call(
        _dinv_kernel,
        grid=(_GRID,),
        in_specs=[pl.BlockSpec((2, _BR, 16), lambda i: (0, i, 0))],
        out_specs=pl.BlockSpec((_BR, _HALF), lambda i: (i, 0)),
        out_shape=jax.ShapeDtypeStruct((_N, _HALF), jnp.float32),
    )(deg16)


def _mm0_kernel(x_ref, w_ref, dv_ref, m0_ref, m1_ref):
    m = jnp.dot(x_ref[...], w_ref[...], preferred_element_type=jnp.float32)
    dv = dv_ref[...]
    m0_ref[...] = m[:, :_HALF] * dv
    m1_ref[...] = m[:, _HALF:] * dv


def _tc_mm0(x, w0, dinv):
    return pl.pallas_call(
        _mm0_kernel,
        grid=(_GRID,),
        in_specs=[
            pl.BlockSpec((_BR, _F), lambda i: (i, 0)),
            pl.BlockSpec((_F, _H), lambda i: (0, 0)),
            pl.BlockSpec((_BR, _HALF), lambda i: (i, 0)),
        ],
        out_specs=(pl.BlockSpec((_BR, _HALF), lambda i: (i, 0)),
                   pl.BlockSpec((_BR, _HALF), lambda i: (i, 0))),
        out_shape=(jax.ShapeDtypeStruct((_N, _HALF), jnp.float32),
                   jax.ShapeDtypeStruct((_N, _HALF), jnp.float32)),
    )(x, w0, dinv)


def _post_kernel(s0_ref, s1_ref, m0_ref, m1_ref, dv_ref, b_ref,
                 r_ref, st_ref, acc1, acc2):
    i = pl.program_id(0)
    dv = dv_ref[...]
    b = b_ref[...]
    p0 = (s0_ref[...] + m0_ref[...]) * dv + b[:, :_HALF]
    p1 = (s1_ref[...] + m1_ref[...]) * dv + b[:, _HALF:]
    r = jnp.concatenate([jnp.maximum(p0, 0.0), jnp.maximum(p1, 0.0)], axis=1)
    r_ref[...] = r

    @pl.when(i == 0)
    def _():
        acc1[...] = jnp.zeros_like(acc1)
        acc2[...] = jnp.zeros_like(acc2)

    acc1[...] += jnp.sum(r, axis=0, keepdims=True)
    acc2[...] += jnp.sum(r * r, axis=0, keepdims=True)

    @pl.when(i == _GRID - 1)
    def _():
        st_ref[...] = jnp.concatenate([acc1[...], acc2[...]], axis=0)


def _tc_post(s0, s1, m0, m1, dinv, b):
    return pl.pallas_call(
        _post_kernel,
        grid=(_GRID,),
        in_specs=[
            pl.BlockSpec((_BR, _HALF), lambda i: (i, 0)),
            pl.BlockSpec((_BR, _HALF), lambda i: (i, 0)),
            pl.BlockSpec((_BR, _HALF), lambda i: (i, 0)),
            pl.BlockSpec((_BR, _HALF), lambda i: (i, 0)),
            pl.BlockSpec((_BR, _HALF), lambda i: (i, 0)),
            pl.BlockSpec((1, _H), lambda i: (0, 0)),
        ],
        out_specs=(pl.BlockSpec((_BR, _H), lambda i: (i, 0)),
                   pl.BlockSpec((2, _H), lambda i: (0, 0))),
        out_shape=(jax.ShapeDtypeStruct((_N, _H), jnp.float32),
                   jax.ShapeDtypeStruct((2, _H), jnp.float32)),
        scratch_shapes=[pltpu.VMEM((1, _H), jnp.float32),
                        pltpu.VMEM((1, _H), jnp.float32)],
    )(s0, s1, m0, m1, dinv, b)


def _mm_kernel(r_ref, st_ref, w_ref, dv_ref, g_ref, bt_ref, m0_ref, m1_ref):
    st = st_ref[...]
    mu = st[0:1] / _N
    var = st[1:2] / _N - mu * mu
    a = g_ref[...] * lax.rsqrt(var + _BN_EPS)
    c = bt_ref[...] - mu * a
    rn = r_ref[...] * a + c
    m = jnp.dot(rn, w_ref[...], preferred_element_type=jnp.float32)
    dv = dv_ref[...]
    m0_ref[...] = m[:, :_HALF] * dv
    m1_ref[...] = m[:, _HALF:] * dv


def _tc_mm(r, st, w, dinv, g, bt):
    return pl.pallas_call(
        _mm_kernel,
        grid=(_GRID,),
        in_specs=[
            pl.BlockSpec((_BR, _H), lambda i: (i, 0)),
            pl.BlockSpec((2, _H), lambda i: (0, 0)),
            pl.BlockSpec((_H, _H), lambda i: (0, 0)),
            pl.BlockSpec((_BR, _HALF), lambda i: (i, 0)),
            pl.BlockSpec((1, _H), lambda i: (0, 0)),
            pl.BlockSpec((1, _H), lambda i: (0, 0)),
        ],
        out_specs=(pl.BlockSpec((_BR, _HALF), lambda i: (i, 0)),
                   pl.BlockSpec((_BR, _HALF), lambda i: (i, 0))),
        out_shape=(jax.ShapeDtypeStruct((_N, _HALF), jnp.float32),
                   jax.ShapeDtypeStruct((_N, _HALF), jnp.float32)),
    )(r, st, w, dinv, g, bt)


def _final_kernel(s0_ref, s1_ref, m0_ref, m1_ref, dv_ref, b_ref, o_ref):
    dv = dv_ref[...]
    b = b_ref[...]
    o0 = (s0_ref[...] + m0_ref[...]) * dv + b[:, :_HALF]
    o1 = (s1_ref[...] + m1_ref[...]) * dv + b[:, _HALF:]
    o_ref[...] = jnp.concatenate([o0, o1], axis=1)


def _tc_final(s0, s1, m0, m1, dinv, b):
    return pl.pallas_call(
        _final_kernel,
        grid=(_GRID,),
        in_specs=[
            pl.BlockSpec((_BR, _HALF), lambda i: (i, 0)),
            pl.BlockSpec((_BR, _HALF), lambda i: (i, 0)),
            pl.BlockSpec((_BR, _HALF), lambda i: (i, 0)),
            pl.BlockSpec((_BR, _HALF), lambda i: (i, 0)),
            pl.BlockSpec((_BR, _HALF), lambda i: (i, 0)),
            pl.BlockSpec((1, _H), lambda i: (0, 0)),
        ],
        out_specs=pl.BlockSpec((_BR, _H), lambda i: (i, 0)),
        out_shape=jax.ShapeDtypeStruct((_N, _H), jnp.float32),
    )(s0, s1, m0, m1, dinv, b)


def kernel(x, edge_index, W0, b0, g0, bt0, W1, b1, g1, bt1, W2, b2, g2, bt2,
           W3, b3, g3, bt3, Wh, bh):
    src = edge_index[0].astype(jnp.int32)
    dst = edge_index[1].astype(jnp.int32)
    src4 = src.reshape(_NSUB, _NCH, _RCH, _W)
    dst4 = dst.reshape(_NSUB, _NCH, _RCH, _W)
    dstd = dst.reshape(2, _NSUB, _NW_D, _WD)
    z16 = jnp.zeros((_N, 16), jnp.float32)
    zrow = jnp.zeros((_N, _HALF), jnp.float32)

    deg16 = _sc_degree(dstd, z16)
    dinv = _tc_dinv(deg16)

    m0, m1 = _tc_mm0(x, W0, dinv)
    layers = [(b0, g0, bt0, W1), (b1, g1, bt1, W2), (b2, g2, bt2, W3)]
    for b, g, bt, wnext in layers:
        s0, s1 = _sc_prop(m0, m1, src4, dst4, zrow)
        r, st = _tc_post(s0, s1, m0, m1, dinv, b.reshape(1, _H))
        m0, m1 = _tc_mm(r, st, wnext, dinv, g.reshape(1, _H), bt.reshape(1, _H))

    s0, s1 = _sc_prop(m0, m1, src4, dst4, zrow)
    r, st = _tc_post(s0, s1, m0, m1, dinv, b3.reshape(1, _H))
    wflat = jnp.transpose(Wh, (1, 0, 2)).reshape(_H, 2 * _C)
    wflat = jnp.pad(wflat, ((0, 0), (0, _H - 2 * _C)))
    m0, m1 = _tc_mm(r, st, wflat, dinv, g3.reshape(1, _H), bt3.reshape(1, _H))

    s0, s1 = _sc_prop(m0, m1, src4, dst4, zrow)
    bh_row = jnp.pad(bh.reshape(1, 2 * _C), ((0, 0), (0, _H - 2 * _C)))
    out = _tc_final(s0, s1, m0, m1, dinv, bh_row)

    return jnp.transpose(out[:, :2 * _C].reshape(_N, _C, 2), (1, 0, 2))
